# baseline (speedup 1.0000x reference)
import functools

import jax
import jax.numpy as jnp
from jax import lax
from jax.experimental import pallas as pl
from jax.experimental.pallas import tpu as pltpu
from jax.experimental.pallas import tpu_sc as plsc

K_TOP = 32
_IDX_BITS = 15
_IDX_MASK = (1 << _IDX_BITS) - 1


def _bitonic_sort_desc(p):
    n = len(p)
    k = 2
    while k <= n:
        jj = k // 2
        while jj >= 1:
            for i in range(n):
                l = i ^ jj
                if l > i:
                    a, b = p[i], p[l]
                    if (i & k) == 0:
                        p[i], p[l] = jnp.maximum(a, b), jnp.minimum(a, b)
                    else:
                        p[i], p[l] = jnp.minimum(a, b), jnp.maximum(a, b)
            jj //= 2
        k *= 2
    return p


def _bitonic_clean_desc(p):
    n = len(p)
    jj = n // 2
    while jj >= 1:
        for i in range(n):
            l = i ^ jj
            if l > i:
                a, b = p[i], p[l]
                p[i], p[l] = jnp.maximum(a, b), jnp.minimum(a, b)
        jj //= 2
    return p


def _encode_topk_kernel(x_ref, bd_ref, w_ref, be_ref, vals_ref, idx_ref,
                        keys_ref, t_ref, *, nj, cj):
    j = pl.program_id(1)
    r = x_ref.shape[0]
    lw = cj // K_TOP
    xl = (x_ref[...] - bd_ref[...]).astype(jnp.bfloat16)
    pre = jnp.dot(xl, w_ref[...], preferred_element_type=jnp.float32)
    pre = pre + be_ref[...]

    bits = lax.bitcast_convert_type(pre, jnp.int32)
    key = jnp.where(bits < 0, bits ^ jnp.int32(0x7FFFFFFF), bits)
    key = (key + jnp.int32(1 << (_IDX_BITS - 1))) & jnp.int32(~_IDX_MASK)
    col = lax.broadcasted_iota(jnp.int32, pre.shape, 1) + j * cj
    keys_ref[...] = (key | (jnp.int32(_IDX_MASK) - col)).reshape(
        r // 8, 8, cj)

    neg = jnp.int32(-(2**31))

    @pl.when(j == 0)
    def _init():
        t_ref[...] = jnp.full(t_ref.shape, neg, jnp.int32)

    def _slab(sb, carry):
        c = [keys_ref[sb, :, pl.ds(s * lw, lw)] for s in range(K_TOP)]
        c = _bitonic_sort_desc(c)
        t = [t_ref[sb, :, pl.ds(s * lw, lw)] for s in range(K_TOP)]
        m = [jnp.maximum(t[i], c[K_TOP - 1 - i]) for i in range(K_TOP)]
        m = _bitonic_clean_desc(m)
        for s in range(K_TOP):
            t_ref[sb, :, pl.ds(s * lw, lw)] = m[s]
        return carry

    lax.fori_loop(0, r // 8, _slab, 0)

    @pl.when(j == nj - 1)
    def _finalize():
        lane_o = lax.broadcasted_iota(jnp.int32, vals_ref.shape, 1)

        def _unpack(mk):
            vb = mk & jnp.int32(~_IDX_MASK)
            fb = jnp.where(vb < 0, vb ^ jnp.int32(0x7FFFFFFF), vb)
            v = jnp.maximum(lax.bitcast_convert_type(fb, jnp.float32), 0.0)
            iid = jnp.int32(_IDX_MASK) - (mk & jnp.int32(_IDX_MASK))
            return v, iid

        nw = 4

        def _extract(t, _):
            tk = t_ref[...]
            ms = []
            for _w in range(nw):
                m = jnp.max(tk, axis=2, keepdims=True)
                ms.append(m)
                tk = jnp.where(tk == m, neg, tk)
            t_ref[...] = tk
            vs = vals_ref[...]
            ids = idx_ref[...]
            for w, m in enumerate(ms):
                v, iid = _unpack(m.reshape(r, 1))
                vs = jnp.where(lane_o == nw * t + w, v, vs)
                ids = jnp.where(lane_o == nw * t + w, iid, ids)
            vals_ref[...] = vs
            idx_ref[...] = ids
            return _

        lax.fori_loop(0, K_TOP // nw, _extract, 0)


def _encode_topk(x, b_dec, W_enc, b_enc):
    b, d_in = x.shape
    d_sae = W_enc.shape[1]
    r = min(b, 512)
    cj = min(d_sae, 4096)
    nj = d_sae // cj
    return pl.pallas_call(
        functools.partial(_encode_topk_kernel, nj=nj, cj=cj),
        grid=(b // r, nj),
        in_specs=[
            pl.BlockSpec((r, d_in), lambda i, j: (i, 0)),
            pl.BlockSpec((1, d_in), lambda i, j: (0, 0)),
            pl.BlockSpec((d_in, cj), lambda i, j: (0, j)),
            pl.BlockSpec((1, cj), lambda i, j: (0, j)),
        ],
        out_specs=[
            pl.BlockSpec((r, K_TOP), lambda i, j: (i, 0)),
            pl.BlockSpec((r, K_TOP), lambda i, j: (i, 0)),
        ],
        out_shape=[
            jax.ShapeDtypeStruct((b, K_TOP), jnp.float32),
            jax.ShapeDtypeStruct((b, K_TOP), jnp.int32),
        ],
        scratch_shapes=[pltpu.VMEM((r // 8, 8, cj), jnp.int32),
                        pltpu.VMEM((r // 8, 8, cj), jnp.int32)],
        compiler_params=pltpu.CompilerParams(
            dimension_semantics=("arbitrary", "arbitrary")),
    )(x, b_dec.reshape(1, -1), W_enc.astype(jnp.bfloat16),
      b_enc.reshape(1, -1))


def _sc_decode(W_dec, idx_flat, vals_flat, b, d_in):
    info = plsc.get_sparse_core_info()
    nc, ns = info.num_cores, info.num_subcores
    nw = nc * ns
    rows_per = b // nw
    nch = d_in // 16

    mesh = plsc.VectorSubcoreMesh(core_axis_name="c", subcore_axis_name="s")

    @functools.partial(
        pl.kernel, mesh=mesh,
        out_type=jax.ShapeDtypeStruct((b, d_in), jnp.float32),
        scratch_types=[
            pltpu.VMEM((rows_per * K_TOP,), jnp.int32),
            pltpu.VMEM((rows_per * K_TOP,), jnp.float32),
            pltpu.VMEM((2, K_TOP, d_in), jnp.float32),
            pltpu.VMEM((d_in,), jnp.float32),
            pltpu.SemaphoreType.DMA((2,)),
        ],
    )
    def dec(wdec_hbm, idx_hbm, vals_hbm, out_hbm, idx_v, vals_v, rows_v,
            out_v, sem):
        wid = lax.axis_index("s") * nc + lax.axis_index("c")
        base = wid * rows_per
        pltpu.sync_copy(idx_hbm.at[pl.ds(base * K_TOP, rows_per * K_TOP)],
                        idx_v)
        pltpu.sync_copy(vals_hbm.at[pl.ds(base * K_TOP, rows_per * K_TOP)],
                        vals_v)

        def _fire(rloc, slot):
            pltpu.async_copy(
                wdec_hbm.at[idx_v.at[pl.ds(rloc * K_TOP, K_TOP)]],
                rows_v.at[slot], sem.at[slot])

        _fire(0, 0)

        def row_body(rloc, carry):
            roff = rloc * K_TOP
            slot = lax.rem(rloc, 2)

            @pl.when(rloc + 1 < rows_per)
            def _prefetch():
                _fire(rloc + 1, 1 - slot)

            pltpu.make_async_copy(
                wdec_hbm.at[pl.ds(0, K_TOP)], rows_v.at[slot],
                sem.at[slot]).wait()
            va = [vals_v[pl.ds(roff + 16 * g, 16)] for g in range(K_TOP // 16)]
            vvecs = [va[kk // 16][kk % 16] for kk in range(K_TOP)]

            def c_body(c, inner):
                co = c * 16
                acc = jnp.zeros((16,), jnp.float32)
                for kk in range(K_TOP):
                    acc = acc + vvecs[kk] * rows_v[slot, kk, pl.ds(co, 16)]
                out_v[pl.ds(co, 16)] = acc
                return inner

            lax.fori_loop(0, nch, c_body, 0)
            pltpu.sync_copy(out_v, out_hbm.at[base + rloc])
            return carry

        lax.fori_loop(0, rows_per, row_body, 0)

    return dec(W_dec, idx_flat, vals_flat)


def _mse_kernel(r_ref, x_ref, bd_ref, o_ref):
    i = pl.program_id(0)
    d = r_ref[...] + bd_ref[...] - x_ref[...]
    s = jnp.sum(d * d).reshape(1, 1)

    @pl.when(i == 0)
    def _init():
        o_ref[...] = jnp.zeros((1, 1), jnp.float32)

    o_ref[...] += s


def _mse(recons, x, b_dec):
    b, d_in = x.shape
    r = min(b, 512)
    out = pl.pallas_call(
        _mse_kernel,
        grid=(b // r,),
        in_specs=[
            pl.BlockSpec((r, d_in), lambda i: (i, 0)),
            pl.BlockSpec((r, d_in), lambda i: (i, 0)),
            pl.BlockSpec((1, d_in), lambda i: (0, 0)),
        ],
        out_specs=pl.BlockSpec((1, 1), lambda i: (0, 0)),
        out_shape=jax.ShapeDtypeStruct((1, 1), jnp.float32),
    )(recons, x, b_dec.reshape(1, -1))
    return out.reshape(())


def kernel(x, W_enc, W_dec, b_enc, b_dec):
    b, d_in = x.shape
    nq = 4 if b % (4 * 512) == 0 else 1
    q = b // nq
    enc = [_encode_topk(x[i * q:(i + 1) * q], b_dec, W_enc, b_enc)
           for i in range(nq)]
    dec = [_sc_decode(W_dec, i_.reshape(-1), v_.reshape(-1), q, d_in)
           for v_, i_ in enc]
    recons = jnp.concatenate(dec, axis=0) if nq > 1 else dec[0]
    return _mse(recons, x, b_dec)


# speedup vs baseline: 1.0064x; 1.0064x over previous
import functools

import jax
import jax.numpy as jnp
from jax import lax
from jax.experimental import pallas as pl
from jax.experimental.pallas import tpu as pltpu
from jax.experimental.pallas import tpu_sc as plsc

K_TOP = 32
_IDX_BITS = 15
_IDX_MASK = (1 << _IDX_BITS) - 1


def _bitonic_sort_desc(p):
    n = len(p)
    k = 2
    while k <= n:
        jj = k // 2
        while jj >= 1:
            for i in range(n):
                l = i ^ jj
                if l > i:
                    a, b = p[i], p[l]
                    if (i & k) == 0:
                        p[i], p[l] = jnp.maximum(a, b), jnp.minimum(a, b)
                    else:
                        p[i], p[l] = jnp.minimum(a, b), jnp.maximum(a, b)
            jj //= 2
        k *= 2
    return p


def _bitonic_clean_desc(p):
    n = len(p)
    jj = n // 2
    while jj >= 1:
        for i in range(n):
            l = i ^ jj
            if l > i:
                a, b = p[i], p[l]
                p[i], p[l] = jnp.maximum(a, b), jnp.minimum(a, b)
        jj //= 2
    return p


def _encode_topk_kernel(x_ref, bd_ref, w_ref, be_ref, vals_ref, idx_ref,
                        keys_ref, t_ref, *, nj, cj):
    j = pl.program_id(1)
    r = x_ref.shape[0]
    lw = cj // K_TOP
    xl = (x_ref[...] - bd_ref[...]).astype(jnp.bfloat16)
    pre = jnp.dot(xl, w_ref[...], preferred_element_type=jnp.float32)
    pre = pre + be_ref[...]

    bits = lax.bitcast_convert_type(pre, jnp.int32)
    key = jnp.where(bits < 0, bits ^ jnp.int32(0x7FFFFFFF), bits)
    key = (key + jnp.int32(1 << (_IDX_BITS - 1))) & jnp.int32(~_IDX_MASK)
    col = lax.broadcasted_iota(jnp.int32, pre.shape, 1) + j * cj
    keys_ref[...] = (key | (jnp.int32(_IDX_MASK) - col)).reshape(
        r // 8, 8, cj)

    neg = jnp.int32(-(2**31))

    @pl.when(j == 0)
    def _init():
        t_ref[...] = jnp.full(t_ref.shape, neg, jnp.int32)

    def _slab(sb, carry):
        c = [keys_ref[sb, :, pl.ds(s * lw, lw)] for s in range(K_TOP)]
        c = _bitonic_sort_desc(c)
        t = [t_ref[sb, :, pl.ds(s * lw, lw)] for s in range(K_TOP)]
        m = [jnp.maximum(t[i], c[K_TOP - 1 - i]) for i in range(K_TOP)]
        m = _bitonic_clean_desc(m)
        for s in range(K_TOP):
            t_ref[sb, :, pl.ds(s * lw, lw)] = m[s]
        return carry

    lax.fori_loop(0, r // 8, _slab, 0, unroll=2)

    @pl.when(j == nj - 1)
    def _finalize():
        lane_o = lax.broadcasted_iota(jnp.int32, vals_ref.shape, 1)

        def _unpack(mk):
            vb = mk & jnp.int32(~_IDX_MASK)
            fb = jnp.where(vb < 0, vb ^ jnp.int32(0x7FFFFFFF), vb)
            v = jnp.maximum(lax.bitcast_convert_type(fb, jnp.float32), 0.0)
            iid = jnp.int32(_IDX_MASK) - (mk & jnp.int32(_IDX_MASK))
            return v, iid

        nw = 4

        def _extract(t, _):
            tk = t_ref[...]
            ms = []
            for _w in range(nw):
                m = jnp.max(tk, axis=2, keepdims=True)
                ms.append(m)
                tk = jnp.where(tk == m, neg, tk)
            t_ref[...] = tk
            vs = vals_ref[...]
            ids = idx_ref[...]
            for w, m in enumerate(ms):
                v, iid = _unpack(m.reshape(r, 1))
                vs = jnp.where(lane_o == nw * t + w, v, vs)
                ids = jnp.where(lane_o == nw * t + w, iid, ids)
            vals_ref[...] = vs
            idx_ref[...] = ids
            return _

        lax.fori_loop(0, K_TOP // nw, _extract, 0)


def _encode_topk(x, b_dec, W_enc, b_enc):
    b, d_in = x.shape
    d_sae = W_enc.shape[1]
    r = min(b, 512)
    cj = min(d_sae, 4096)
    nj = d_sae // cj
    return pl.pallas_call(
        functools.partial(_encode_topk_kernel, nj=nj, cj=cj),
        grid=(b // r, nj),
        in_specs=[
            pl.BlockSpec((r, d_in), lambda i, j: (i, 0)),
            pl.BlockSpec((1, d_in), lambda i, j: (0, 0)),
            pl.BlockSpec((d_in, cj), lambda i, j: (0, j)),
            pl.BlockSpec((1, cj), lambda i, j: (0, j)),
        ],
        out_specs=[
            pl.BlockSpec((r, K_TOP), lambda i, j: (i, 0)),
            pl.BlockSpec((r, K_TOP), lambda i, j: (i, 0)),
        ],
        out_shape=[
            jax.ShapeDtypeStruct((b, K_TOP), jnp.float32),
            jax.ShapeDtypeStruct((b, K_TOP), jnp.int32),
        ],
        scratch_shapes=[pltpu.VMEM((r // 8, 8, cj), jnp.int32),
                        pltpu.VMEM((r // 8, 8, cj), jnp.int32)],
        compiler_params=pltpu.CompilerParams(
            dimension_semantics=("arbitrary", "arbitrary")),
    )(x, b_dec.reshape(1, -1), W_enc.astype(jnp.bfloat16),
      b_enc.reshape(1, -1))


def _sc_decode(W_dec, idx_flat, vals_flat, b, d_in):
    info = plsc.get_sparse_core_info()
    nc, ns = info.num_cores, info.num_subcores
    nw = nc * ns
    rows_per = b // nw
    nch = d_in // 16

    mesh = plsc.VectorSubcoreMesh(core_axis_name="c", subcore_axis_name="s")

    @functools.partial(
        pl.kernel, mesh=mesh,
        out_type=jax.ShapeDtypeStruct((b, d_in), jnp.float32),
        scratch_types=[
            pltpu.VMEM((rows_per * K_TOP,), jnp.int32),
            pltpu.VMEM((rows_per * K_TOP,), jnp.float32),
            pltpu.VMEM((2, K_TOP, d_in), jnp.float32),
            pltpu.VMEM((d_in,), jnp.float32),
            pltpu.SemaphoreType.DMA((2,)),
        ],
    )
    def dec(wdec_hbm, idx_hbm, vals_hbm, out_hbm, idx_v, vals_v, rows_v,
            out_v, sem):
        wid = lax.axis_index("s") * nc + lax.axis_index("c")
        base = wid * rows_per
        pltpu.sync_copy(idx_hbm.at[pl.ds(base * K_TOP, rows_per * K_TOP)],
                        idx_v)
        pltpu.sync_copy(vals_hbm.at[pl.ds(base * K_TOP, rows_per * K_TOP)],
                        vals_v)

        def _fire(rloc, slot):
            pltpu.async_copy(
                wdec_hbm.at[idx_v.at[pl.ds(rloc * K_TOP, K_TOP)]],
                rows_v.at[slot], sem.at[slot])

        _fire(0, 0)

        def row_body(rloc, carry):
            roff = rloc * K_TOP
            slot = lax.rem(rloc, 2)

            @pl.when(rloc + 1 < rows_per)
            def _prefetch():
                _fire(rloc + 1, 1 - slot)

            pltpu.make_async_copy(
                wdec_hbm.at[pl.ds(0, K_TOP)], rows_v.at[slot],
                sem.at[slot]).wait()
            va = [vals_v[pl.ds(roff + 16 * g, 16)] for g in range(K_TOP // 16)]
            vvecs = [va[kk // 16][kk % 16] for kk in range(K_TOP)]

            def c_body(c, inner):
                co = c * 16
                acc = jnp.zeros((16,), jnp.float32)
                for kk in range(K_TOP):
                    acc = acc + vvecs[kk] * rows_v[slot, kk, pl.ds(co, 16)]
                out_v[pl.ds(co, 16)] = acc
                return inner

            lax.fori_loop(0, nch, c_body, 0)
            pltpu.sync_copy(out_v, out_hbm.at[base + rloc])
            return carry

        lax.fori_loop(0, rows_per, row_body, 0)

    return dec(W_dec, idx_flat, vals_flat)


def _mse_kernel(r_ref, x_ref, bd_ref, o_ref):
    i = pl.program_id(0)
    d = r_ref[...] + bd_ref[...] - x_ref[...]
    s = jnp.sum(d * d).reshape(1, 1)

    @pl.when(i == 0)
    def _init():
        o_ref[...] = jnp.zeros((1, 1), jnp.float32)

    o_ref[...] += s


def _mse(recons, x, b_dec):
    b, d_in = x.shape
    r = min(b, 512)
    out = pl.pallas_call(
        _mse_kernel,
        grid=(b // r,),
        in_specs=[
            pl.BlockSpec((r, d_in), lambda i: (i, 0)),
            pl.BlockSpec((r, d_in), lambda i: (i, 0)),
            pl.BlockSpec((1, d_in), lambda i: (0, 0)),
        ],
        out_specs=pl.BlockSpec((1, 1), lambda i: (0, 0)),
        out_shape=jax.ShapeDtypeStruct((1, 1), jnp.float32),
    )(recons, x, b_dec.reshape(1, -1))
    return out.reshape(())


def kernel(x, W_enc, W_dec, b_enc, b_dec):
    b, d_in = x.shape
    nq = 4 if b % (4 * 512) == 0 else 1
    q = b // nq
    enc = [_encode_topk(x[i * q:(i + 1) * q], b_dec, W_enc, b_enc)
           for i in range(nq)]
    dec = [_sc_decode(W_dec, i_.reshape(-1), v_.reshape(-1), q, d_in)
           for v_, i_ in enc]
    recons = jnp.concatenate(dec, axis=0) if nq > 1 else dec[0]
    return _mse(recons, x, b_dec)


# speedup vs baseline: 1.0066x; 1.0002x over previous
import functools

import jax
import jax.numpy as jnp
from jax import lax
from jax.experimental import pallas as pl
from jax.experimental.pallas import tpu as pltpu
from jax.experimental.pallas import tpu_sc as plsc

K_TOP = 32
_IDX_BITS = 15
_IDX_MASK = (1 << _IDX_BITS) - 1


def _bitonic_sort_desc(p):
    n = len(p)
    k = 2
    while k <= n:
        jj = k // 2
        while jj >= 1:
            for i in range(n):
                l = i ^ jj
                if l > i:
                    a, b = p[i], p[l]
                    if (i & k) == 0:
                        p[i], p[l] = jnp.maximum(a, b), jnp.minimum(a, b)
                    else:
                        p[i], p[l] = jnp.minimum(a, b), jnp.maximum(a, b)
            jj //= 2
        k *= 2
    return p


def _bitonic_clean_desc(p):
    n = len(p)
    jj = n // 2
    while jj >= 1:
        for i in range(n):
            l = i ^ jj
            if l > i:
                a, b = p[i], p[l]
                p[i], p[l] = jnp.maximum(a, b), jnp.minimum(a, b)
        jj //= 2
    return p


def _encode_topk_kernel(x_ref, bd_ref, w_ref, be_ref, vals_ref, idx_ref,
                        keys_ref, t_ref, *, nj, cj):
    j = pl.program_id(1)
    r = x_ref.shape[0]
    lw = cj // K_TOP
    xl = (x_ref[...] - bd_ref[...]).astype(jnp.bfloat16)
    pre = jnp.dot(xl, w_ref[...], preferred_element_type=jnp.float32)
    pre = pre + be_ref[...]

    bits = lax.bitcast_convert_type(pre, jnp.int32)
    key = jnp.where(bits < 0, bits ^ jnp.int32(0x7FFFFFFF), bits)
    key = (key + jnp.int32(1 << (_IDX_BITS - 1))) & jnp.int32(~_IDX_MASK)
    col = lax.broadcasted_iota(jnp.int32, pre.shape, 1) + j * cj
    keys_ref[...] = (key | (jnp.int32(_IDX_MASK) - col)).reshape(
        r // 8, 8, cj)

    neg = jnp.int32(-(2**31))

    @pl.when(j == 0)
    def _init():
        t_ref[...] = jnp.full(t_ref.shape, neg, jnp.int32)

    def _slab(sb, carry):
        c = [keys_ref[sb, :, pl.ds(s * lw, lw)] for s in range(K_TOP)]
        c = _bitonic_sort_desc(c)
        t = [t_ref[sb, :, pl.ds(s * lw, lw)] for s in range(K_TOP)]
        m = [jnp.maximum(t[i], c[K_TOP - 1 - i]) for i in range(K_TOP)]
        m = _bitonic_clean_desc(m)
        for s in range(K_TOP):
            t_ref[sb, :, pl.ds(s * lw, lw)] = m[s]
        return carry

    lax.fori_loop(0, r // 8, _slab, 0, unroll=2)

    @pl.when(j == nj - 1)
    def _finalize():
        lane_o = lax.broadcasted_iota(jnp.int32, vals_ref.shape, 1)

        def _unpack(mk):
            vb = mk & jnp.int32(~_IDX_MASK)
            fb = jnp.where(vb < 0, vb ^ jnp.int32(0x7FFFFFFF), vb)
            v = jnp.maximum(lax.bitcast_convert_type(fb, jnp.float32), 0.0)
            iid = jnp.int32(_IDX_MASK) - (mk & jnp.int32(_IDX_MASK))
            return v, iid

        nw = 4

        def _extract(t, _):
            tk = t_ref[...]
            ms = []
            for _w in range(nw):
                m = jnp.max(tk, axis=2, keepdims=True)
                ms.append(m)
                tk = jnp.where(tk == m, neg, tk)
            t_ref[...] = tk
            vs = vals_ref[...]
            ids = idx_ref[...]
            for w, m in enumerate(ms):
                v, iid = _unpack(m.reshape(r, 1))
                vs = jnp.where(lane_o == nw * t + w, v, vs)
                ids = jnp.where(lane_o == nw * t + w, iid, ids)
            vals_ref[...] = vs
            idx_ref[...] = ids
            return _

        lax.fori_loop(0, K_TOP // nw, _extract, 0, unroll=2)


def _encode_topk(x, b_dec, W_enc, b_enc):
    b, d_in = x.shape
    d_sae = W_enc.shape[1]
    r = min(b, 512)
    cj = min(d_sae, 4096)
    nj = d_sae // cj
    return pl.pallas_call(
        functools.partial(_encode_topk_kernel, nj=nj, cj=cj),
        grid=(b // r, nj),
        in_specs=[
            pl.BlockSpec((r, d_in), lambda i, j: (i, 0)),
            pl.BlockSpec((1, d_in), lambda i, j: (0, 0)),
            pl.BlockSpec((d_in, cj), lambda i, j: (0, j)),
            pl.BlockSpec((1, cj), lambda i, j: (0, j)),
        ],
        out_specs=[
            pl.BlockSpec((r, K_TOP), lambda i, j: (i, 0)),
            pl.BlockSpec((r, K_TOP), lambda i, j: (i, 0)),
        ],
        out_shape=[
            jax.ShapeDtypeStruct((b, K_TOP), jnp.float32),
            jax.ShapeDtypeStruct((b, K_TOP), jnp.int32),
        ],
        scratch_shapes=[pltpu.VMEM((r // 8, 8, cj), jnp.int32),
                        pltpu.VMEM((r // 8, 8, cj), jnp.int32)],
        compiler_params=pltpu.CompilerParams(
            dimension_semantics=("arbitrary", "arbitrary")),
    )(x, b_dec.reshape(1, -1), W_enc.astype(jnp.bfloat16),
      b_enc.reshape(1, -1))


def _sc_decode(W_dec, idx_flat, vals_flat, b, d_in):
    info = plsc.get_sparse_core_info()
    nc, ns = info.num_cores, info.num_subcores
    nw = nc * ns
    rows_per = b // nw
    nch = d_in // 16

    mesh = plsc.VectorSubcoreMesh(core_axis_name="c", subcore_axis_name="s")

    @functools.partial(
        pl.kernel, mesh=mesh,
        out_type=jax.ShapeDtypeStruct((b, d_in), jnp.float32),
        scratch_types=[
            pltpu.VMEM((rows_per * K_TOP,), jnp.int32),
            pltpu.VMEM((rows_per * K_TOP,), jnp.float32),
            pltpu.VMEM((2, K_TOP, d_in), jnp.float32),
            pltpu.VMEM((d_in,), jnp.float32),
            pltpu.SemaphoreType.DMA((2,)),
        ],
    )
    def dec(wdec_hbm, idx_hbm, vals_hbm, out_hbm, idx_v, vals_v, rows_v,
            out_v, sem):
        wid = lax.axis_index("s") * nc + lax.axis_index("c")
        base = wid * rows_per
        pltpu.sync_copy(idx_hbm.at[pl.ds(base * K_TOP, rows_per * K_TOP)],
                        idx_v)
        pltpu.sync_copy(vals_hbm.at[pl.ds(base * K_TOP, rows_per * K_TOP)],
                        vals_v)

        def _fire(rloc, slot):
            pltpu.async_copy(
                wdec_hbm.at[idx_v.at[pl.ds(rloc * K_TOP, K_TOP)]],
                rows_v.at[slot], sem.at[slot])

        _fire(0, 0)

        def row_body(rloc, carry):
            roff = rloc * K_TOP
            slot = lax.rem(rloc, 2)

            @pl.when(rloc + 1 < rows_per)
            def _prefetch():
                _fire(rloc + 1, 1 - slot)

            pltpu.make_async_copy(
                wdec_hbm.at[pl.ds(0, K_TOP)], rows_v.at[slot],
                sem.at[slot]).wait()
            va = [vals_v[pl.ds(roff + 16 * g, 16)] for g in range(K_TOP // 16)]
            vvecs = [va[kk // 16][kk % 16] for kk in range(K_TOP)]

            def c_body(c, inner):
                co = c * 16
                acc = jnp.zeros((16,), jnp.float32)
                for kk in range(K_TOP):
                    acc = acc + vvecs[kk] * rows_v[slot, kk, pl.ds(co, 16)]
                out_v[pl.ds(co, 16)] = acc
                return inner

            lax.fori_loop(0, nch, c_body, 0)
            pltpu.sync_copy(out_v, out_hbm.at[base + rloc])
            return carry

        lax.fori_loop(0, rows_per, row_body, 0)

    return dec(W_dec, idx_flat, vals_flat)


def _mse_kernel(r_ref, x_ref, bd_ref, o_ref):
    i = pl.program_id(0)
    d = r_ref[...] + bd_ref[...] - x_ref[...]
    s = jnp.sum(d * d).reshape(1, 1)

    @pl.when(i == 0)
    def _init():
        o_ref[...] = jnp.zeros((1, 1), jnp.float32)

    o_ref[...] += s


def _mse(recons, x, b_dec):
    b, d_in = x.shape
    r = min(b, 512)
    out = pl.pallas_call(
        _mse_kernel,
        grid=(b // r,),
        in_specs=[
            pl.BlockSpec((r, d_in), lambda i: (i, 0)),
            pl.BlockSpec((r, d_in), lambda i: (i, 0)),
            pl.BlockSpec((1, d_in), lambda i: (0, 0)),
        ],
        out_specs=pl.BlockSpec((1, 1), lambda i: (0, 0)),
        out_shape=jax.ShapeDtypeStruct((1, 1), jnp.float32),
    )(recons, x, b_dec.reshape(1, -1))
    return out.reshape(())


def kernel(x, W_enc, W_dec, b_enc, b_dec):
    b, d_in = x.shape
    nq = 8 if b % (8 * 512) == 0 else 1
    q = b // nq
    enc = [_encode_topk(x[i * q:(i + 1) * q], b_dec, W_enc, b_enc)
           for i in range(nq)]
    dec = [_sc_decode(W_dec, i_.reshape(-1), v_.reshape(-1), q, d_in)
           for v_, i_ in enc]
    recons = jnp.concatenate(dec, axis=0) if nq > 1 else dec[0]
    return _mse(recons, x, b_dec)
